# baseline (speedup 1.0000x reference)
import functools

import jax
import jax.numpy as jnp
from jax import lax
from jax.experimental import pallas as pl
from jax.experimental.pallas import tpu as pltpu
from jax.experimental.pallas import tpu_sc as plsc

_BATCH = 4096
_HIST = 50
_D = 64
_B = _BATCH * _HIST
_NW = 32
_B_PER_W = _B // _NW
_CHUNK = 800
_NCHUNK = _B_PER_W // _CHUNK


def _gather_body(idx_hbm, table_hbm, out_hbm, idx_v, rows0, rows1, sem0, sem1):
    wid = lax.axis_index("s") * 2 + lax.axis_index("c")
    base = wid * _B_PER_W
    pltpu.sync_copy(idx_hbm.at[pl.ds(base, _B_PER_W)], idx_v)
    bufs = (rows0, rows1)
    sems = (sem0, sem1)
    copies = [None, None]
    copies[0] = pltpu.async_copy(
        table_hbm.at[idx_v.at[pl.ds(0, _CHUNK)]], bufs[0], sems[0]
    )
    for c in range(_NCHUNK):
        b = c % 2
        copies[b].wait()
        if c + 1 < _NCHUNK:
            nb = (c + 1) % 2
            copies[nb] = pltpu.async_copy(
                table_hbm.at[idx_v.at[pl.ds((c + 1) * _CHUNK, _CHUNK)]],
                bufs[nb],
                sems[nb],
            )
        pltpu.sync_copy(bufs[b], out_hbm.at[pl.ds(base + c * _CHUNK, _CHUNK)])


@jax.jit
def _sc_gather(idx_flat, entity_table):
    mesh = plsc.VectorSubcoreMesh(core_axis_name="c", subcore_axis_name="s")
    fn = functools.partial(
        pl.kernel,
        mesh=mesh,
        out_type=jax.ShapeDtypeStruct((_B, _D), jnp.float32),
        scratch_types=[
            pltpu.VMEM((_B_PER_W,), jnp.int32),
            pltpu.VMEM((_CHUNK, _D), jnp.float32),
            pltpu.VMEM((_CHUNK, _D), jnp.float32),
            pltpu.SemaphoreType.DMA,
            pltpu.SemaphoreType.DMA,
        ],
        compiler_params=pltpu.CompilerParams(use_tc_tiling_on_sc=False),
    )(_gather_body)
    return fn(idx_flat, entity_table)


def kernel(inputs, entity_table, relation_table):
    idx_flat = inputs.reshape(_B).astype(jnp.int32)
    out = _sc_gather(idx_flat, entity_table)
    return out.reshape(_BATCH, _HIST, _D)


# speedup vs baseline: 1.2493x; 1.2493x over previous
import functools

import jax
import jax.numpy as jnp
from jax import lax
from jax.experimental import pallas as pl
from jax.experimental.pallas import tpu as pltpu
from jax.experimental.pallas import tpu_sc as plsc

_BATCH = 4096
_HIST = 50
_D = 64
_DP = 128
_B = _BATCH * _HIST
_NV = 1000000
_NW = 32
_B_PER_W = _B // _NW
_CHUNK = 400
_NCHUNK = _B_PER_W // _CHUNK

_TCW = 7936
_TC_GRID = 126
_NMAIN = _TCW * _TC_GRID


def _repack_body(tt_hbm, out_hbm, vin, vout, sin, sout):
    g = pl.program_id(0)
    s = lax.rem(g, 2)

    def in_copy(blk, slot):
        return pltpu.make_async_copy(
            tt_hbm.at[:, pl.ds(blk * _TCW, _TCW)], vin.at[slot], sin.at[slot]
        )

    def out_copy(blk, slot):
        return pltpu.make_async_copy(
            vout.at[slot], out_hbm.at[pl.ds(blk * _TCW, _TCW)], sout.at[slot]
        )

    @pl.when(g == 0)
    def _():
        in_copy(0, 0).start()
        in_copy(1, 1).start()

    in_copy(g, s).wait()
    eye = (
        lax.broadcasted_iota(jnp.int32, (_D, _D), 0)
        == lax.broadcasted_iota(jnp.int32, (_D, _D), 1)
    ).astype(jnp.float32)
    t = lax.dot_general(
        vin[s],
        eye,
        dimension_numbers=(((0,), (0,)), ((), ())),
        preferred_element_type=jnp.float32,
    )
    vout[s] = jnp.concatenate([t, jnp.zeros((_TCW, _D), jnp.float32)], axis=1)

    @pl.when(g >= 2)
    def _():
        out_copy(g - 2, s).wait()

    out_copy(g, s).start()

    @pl.when(g + 2 < _TC_GRID)
    def _():
        in_copy(g + 2, s).start()

    @pl.when(g == _TC_GRID - 1)
    def _():
        out_copy(g - 1, lax.rem(g + 1, 2)).wait()
        out_copy(g, s).wait()


@jax.jit
def _tc_repack(tt):
    return pl.pallas_call(
        _repack_body,
        grid=(_TC_GRID,),
        in_specs=[pl.BlockSpec(memory_space=pl.ANY)],
        out_specs=pl.BlockSpec(memory_space=pl.ANY),
        out_shape=jax.ShapeDtypeStruct((_NV, _DP), jnp.float32),
        scratch_shapes=[
            pltpu.VMEM((2, _D, _TCW), jnp.float32),
            pltpu.VMEM((2, _TCW, _DP), jnp.float32),
            pltpu.SemaphoreType.DMA((2,)),
            pltpu.SemaphoreType.DMA((2,)),
        ],
    )(tt)


def _gather_body(idx_hbm, table_hbm, out_hbm, idx_v, rows0, rows1, sem0, sem1):
    wid = lax.axis_index("s") * 2 + lax.axis_index("c")
    base = wid * _B_PER_W
    pltpu.sync_copy(idx_hbm.at[pl.ds(base, _B_PER_W)], idx_v)
    bufs = (rows0, rows1)
    sems = (sem0, sem1)
    copies = [None, None]
    copies[0] = pltpu.async_copy(
        table_hbm.at[idx_v.at[pl.ds(0, _CHUNK)]], bufs[0], sems[0]
    )
    for c in range(_NCHUNK):
        b = c % 2
        copies[b].wait()
        if c + 1 < _NCHUNK:
            nb = (c + 1) % 2
            copies[nb] = pltpu.async_copy(
                table_hbm.at[idx_v.at[pl.ds((c + 1) * _CHUNK, _CHUNK)]],
                bufs[nb],
                sems[nb],
            )
        pltpu.sync_copy(bufs[b], out_hbm.at[pl.ds(base + c * _CHUNK, _CHUNK)])


@jax.jit
def _sc_gather(idx_flat, table128):
    mesh = plsc.VectorSubcoreMesh(core_axis_name="c", subcore_axis_name="s")
    fn = functools.partial(
        pl.kernel,
        mesh=mesh,
        out_type=jax.ShapeDtypeStruct((_B, _DP), jnp.float32),
        scratch_types=[
            pltpu.VMEM((_B_PER_W,), jnp.int32),
            pltpu.VMEM((_CHUNK, _DP), jnp.float32),
            pltpu.VMEM((_CHUNK, _DP), jnp.float32),
            pltpu.SemaphoreType.DMA,
            pltpu.SemaphoreType.DMA,
        ],
        compiler_params=pltpu.CompilerParams(use_tc_tiling_on_sc=True),
    )(_gather_body)
    return fn(idx_flat, table128)


def kernel(inputs, entity_table, relation_table):
    idx_flat = inputs.reshape(_B).astype(jnp.int32)
    table128 = _tc_repack(entity_table.T)
    out128 = _sc_gather(idx_flat, table128)
    out_main = out128[:, :_D]
    tail = entity_table[_NMAIN:]
    in_tail = idx_flat >= _NMAIN
    tail_idx = jnp.where(in_tail, idx_flat - _NMAIN, 0)
    onehot = jax.nn.one_hot(tail_idx, _NV - _NMAIN, dtype=jnp.float32)
    fix = onehot @ tail
    out = jnp.where(in_tail[:, None], fix, out_main)
    return out.reshape(_BATCH, _HIST, _D)


# speedup vs baseline: 1.5262x; 1.2216x over previous
import functools

import jax
import jax.numpy as jnp
from jax import lax
from jax.experimental import pallas as pl
from jax.experimental.pallas import tpu as pltpu
from jax.experimental.pallas import tpu_sc as plsc

_BATCH = 4096
_HIST = 50
_D = 64
_DP = 128
_B = _BATCH * _HIST
_NV = 1000000
_NW = 32
_B_PER_W = _B // _NW
_CHUNK = 400
_NCHUNK = _B_PER_W // _CHUNK

_TCW = 7936
_TC_GRID = 126
_NMAIN = _TCW * _TC_GRID


def _repack_body(tt_hbm, tail_hbm, out_hbm, vin, vout, vtin, vtout, sin, sout, stail):
    g = pl.program_id(0)
    s = lax.rem(g, 2)

    def in_copy(blk, slot):
        return pltpu.make_async_copy(
            tt_hbm.at[:, pl.ds(blk * _TCW, _TCW)], vin.at[slot], sin.at[slot]
        )

    def out_copy(blk, slot):
        return pltpu.make_async_copy(
            vout.at[slot], out_hbm.at[pl.ds(blk * _TCW, _TCW)], sout.at[slot]
        )

    @pl.when(g == 0)
    def _():
        in_copy(0, 0).start()
        in_copy(1, 1).start()

    in_copy(g, s).wait()
    eye = (
        lax.broadcasted_iota(jnp.int32, (_D, _D), 0)
        == lax.broadcasted_iota(jnp.int32, (_D, _D), 1)
    ).astype(jnp.float32)
    t = lax.dot_general(
        vin[s],
        eye,
        dimension_numbers=(((0,), (0,)), ((), ())),
        preferred_element_type=jnp.float32,
    )
    vout[s] = jnp.concatenate([t, jnp.zeros((_TCW, _D), jnp.float32)], axis=1)

    @pl.when(g >= 2)
    def _():
        out_copy(g - 2, s).wait()

    out_copy(g, s).start()

    @pl.when(g + 2 < _TC_GRID)
    def _():
        in_copy(g + 2, s).start()

    @pl.when(g == _TC_GRID - 1)
    def _():
        tin = pltpu.make_async_copy(tail_hbm, vtin, stail.at[0])
        tin.start()
        tin.wait()
        vtout[...] = jnp.concatenate(
            [vtin[...], jnp.zeros((_NV - _NMAIN, _D), jnp.float32)], axis=1
        )
        tout = pltpu.make_async_copy(
            vtout, out_hbm.at[pl.ds(_NMAIN, _NV - _NMAIN)], stail.at[1]
        )
        tout.start()
        out_copy(g - 1, lax.rem(g + 1, 2)).wait()
        out_copy(g, s).wait()
        tout.wait()


@jax.jit
def _tc_repack(tt, tail):
    return pl.pallas_call(
        _repack_body,
        grid=(_TC_GRID,),
        in_specs=[
            pl.BlockSpec(memory_space=pl.ANY),
            pl.BlockSpec(memory_space=pl.ANY),
        ],
        out_specs=pl.BlockSpec(memory_space=pl.ANY),
        out_shape=jax.ShapeDtypeStruct((_NV, _DP), jnp.float32),
        scratch_shapes=[
            pltpu.VMEM((2, _D, _TCW), jnp.float32),
            pltpu.VMEM((2, _TCW, _DP), jnp.float32),
            pltpu.VMEM((_NV - _NMAIN, _D), jnp.float32),
            pltpu.VMEM((_NV - _NMAIN, _DP), jnp.float32),
            pltpu.SemaphoreType.DMA((2,)),
            pltpu.SemaphoreType.DMA((2,)),
            pltpu.SemaphoreType.DMA((2,)),
        ],
    )(tt, tail)


def _gather_body(idx_hbm, table_hbm, out_hbm, idx_v, rows0, rows1, sem0, sem1):
    wid = lax.axis_index("s") * 2 + lax.axis_index("c")
    base = wid * _B_PER_W
    pltpu.sync_copy(idx_hbm.at[pl.ds(base, _B_PER_W)], idx_v)
    bufs = (rows0, rows1)
    sems = (sem0, sem1)
    copies = [None, None]
    copies[0] = pltpu.async_copy(
        table_hbm.at[idx_v.at[pl.ds(0, _CHUNK)]], bufs[0], sems[0]
    )
    for c in range(_NCHUNK):
        b = c % 2
        copies[b].wait()
        if c + 1 < _NCHUNK:
            nb = (c + 1) % 2
            copies[nb] = pltpu.async_copy(
                table_hbm.at[idx_v.at[pl.ds((c + 1) * _CHUNK, _CHUNK)]],
                bufs[nb],
                sems[nb],
            )
        pltpu.sync_copy(bufs[b], out_hbm.at[pl.ds(base + c * _CHUNK, _CHUNK)])


@jax.jit
def _sc_gather(idx_flat, table128):
    mesh = plsc.VectorSubcoreMesh(core_axis_name="c", subcore_axis_name="s")
    fn = functools.partial(
        pl.kernel,
        mesh=mesh,
        out_type=jax.ShapeDtypeStruct((_B, _DP), jnp.float32),
        scratch_types=[
            pltpu.VMEM((_B_PER_W,), jnp.int32),
            pltpu.VMEM((_CHUNK, _DP), jnp.float32),
            pltpu.VMEM((_CHUNK, _DP), jnp.float32),
            pltpu.SemaphoreType.DMA,
            pltpu.SemaphoreType.DMA,
        ],
        compiler_params=pltpu.CompilerParams(use_tc_tiling_on_sc=True),
    )(_gather_body)
    return fn(idx_flat, table128)


def kernel(inputs, entity_table, relation_table):
    idx_flat = inputs.reshape(_B).astype(jnp.int32)
    table128 = _tc_repack(entity_table.T, entity_table[_NMAIN:])
    out128 = _sc_gather(idx_flat, table128)
    return out128[:, :_D].reshape(_BATCH, _HIST, _D)
